# baseline (speedup 1.0000x reference)
import functools

import jax
import jax.numpy as jnp
from jax import lax
from jax.experimental import pallas as pl
from jax.experimental.pallas import tpu as pltpu
from jax.experimental.pallas import tpu_sc as plsc

B = 4096
L = 200
D = 64
HALF = L // 2
NC = 2
NS = 16
NW = NC * NS
SEQ_PER_W = B // NW

_mesh = plsc.VectorSubcoreMesh(core_axis_name="c", subcore_axis_name="s")


@functools.partial(
    pl.kernel,
    out_type=jax.ShapeDtypeStruct((B, L, D), jnp.float32),
    mesh=_mesh,
    compiler_params=pltpu.CompilerParams(use_tc_tiling_on_sc=False),
    scratch_types=[
        pltpu.VMEM((L, D), jnp.float32),
        pltpu.VMEM((2, HALF), jnp.int32),
        pltpu.VMEM((L, D), jnp.float32),
        pltpu.SemaphoreType.DMA,
    ],
)
def _emb(seq_hbm, table_hbm, pos_hbm, out_hbm, pos_v, idx_v, rows_v, sem):
    wid = lax.axis_index("s") * NC + lax.axis_index("c")
    pltpu.sync_copy(pos_hbm, pos_v)

    def body(i, carry):
        b = wid * SEQ_PER_W + i
        pltpu.sync_copy(seq_hbm.at[pl.ds(2 * b, 2)], idx_v)
        cp0 = pltpu.async_copy(
            table_hbm.at[idx_v.at[0]], rows_v.at[pl.ds(0, HALF)], sem)
        cp1 = pltpu.async_copy(
            table_hbm.at[idx_v.at[1]], rows_v.at[pl.ds(HALF, HALF)], sem)
        cp0.wait()
        cp1.wait()

        def add_row(l, c2):
            for c in range(D // 16):
                sl = pl.ds(c * 16, 16)
                rows_v[l, sl] = rows_v[l, sl] + pos_v[l, sl]
            return c2

        lax.fori_loop(0, L, add_row, 0)
        pltpu.sync_copy(rows_v, out_hbm.at[b])
        return carry

    lax.fori_loop(0, SEQ_PER_W, body, 0)


def kernel(seq, table, pos_table):
    seq2 = seq.reshape(2 * B, HALF).astype(jnp.int32)
    return _emb(seq2, table, pos_table)
